# Initial kernel scaffold; baseline (speedup 1.0000x reference)
import jax
import jax.numpy as jnp
from jax.experimental import pallas as pl


def kernel(hidden_states, weight):
    raise NotImplementedError("write your pallas kernel here")


# speedup vs baseline: 1.7603x; 1.7603x over previous
import functools

import jax
import jax.numpy as jnp
from jax.experimental import pallas as pl
from jax.experimental.pallas import tpu as pltpu

_EPS = 1e-06
_TWO_N = 65536.0
_INV_TWO_N = 1.0 / 65536.0


def _quantize(x):
    x32 = jnp.clip(x.astype(jnp.float32), -500.0, 500.0)
    signs = jnp.sign(x32)
    a = jnp.abs(x32)
    is_low = a < 10.0
    v_max = jnp.where(is_low, 10.0, 50.0)
    t = a / v_max * _TWO_N
    f = jnp.floor(t)
    q = f * _INV_TWO_N * v_max
    cap = v_max * (1.0 - _INV_TWO_N)
    return jnp.minimum(q, cap) * signs


def _rows_kernel(x_ref, w_ref, o_ref):
    h = _quantize(x_ref[...])
    variance = jnp.mean(h * h, axis=-1, keepdims=True)
    o_ref[...] = (h * jax.lax.rsqrt(variance + _EPS)) * w_ref[...]


@functools.partial(jax.jit, static_argnames=("block_rows",))
def _spike_ln(x2d, weight, block_rows=256):
    rows = x2d.shape[0]
    hidden = x2d.shape[1]
    grid = (rows // block_rows,)
    return pl.pallas_call(
        _rows_kernel,
        grid=grid,
        in_specs=[
            pl.BlockSpec((block_rows, hidden), lambda i: (i, 0)),
            pl.BlockSpec((1, hidden), lambda i: (0, 0)),
        ],
        out_specs=pl.BlockSpec((block_rows, hidden), lambda i: (i, 0)),
        out_shape=jax.ShapeDtypeStruct((rows, hidden), x2d.dtype),
    )(x2d, weight)


def kernel(hidden_states, weight):
    input_dtype = hidden_states.dtype
    b, s, hidden = hidden_states.shape
    x2d = hidden_states.reshape(b * s, hidden)
    out = _spike_ln(x2d, weight.reshape(1, hidden).astype(jnp.float32))
    return out.reshape(b, s, hidden).astype(input_dtype)
